# Initial kernel scaffold; baseline (speedup 1.0000x reference)
import jax
import jax.numpy as jnp
from jax.experimental import pallas as pl


def kernel(x, table):
    raise NotImplementedError("write your pallas kernel here")


# speedup vs baseline: 1.5476x; 1.5476x over previous
import functools

import jax
import jax.numpy as jnp
from jax import lax
from jax.experimental import pallas as pl
from jax.experimental.pallas import tpu as pltpu
from jax.experimental.pallas import tpu_sc as plsc

EMBED = 32
LANES = 128
G = 8
CHUNK = G * LANES


def kernel(x, table):
    idx = x.reshape(-1).astype(jnp.int32)
    n = idx.shape[0]
    idx2 = idx.reshape(n // LANES, LANES)

    mesh = plsc.VectorSubcoreMesh(core_axis_name="c", subcore_axis_name="s")
    nw = mesh.num_cores * mesh.num_subcores
    rows_per_w = (n // LANES) // nw
    nch = rows_per_w // G

    @functools.partial(
        pl.kernel,
        out_type=jax.ShapeDtypeStruct((n, EMBED), jnp.float32),
        mesh=mesh,
        scratch_types=[
            pltpu.VMEM((G, LANES), jnp.int32),
            pltpu.VMEM((CHUNK, EMBED), jnp.float32),
            pltpu.SemaphoreType.DMA,
        ],
        compiler_params=pltpu.CompilerParams(use_tc_tiling_on_sc=False),
    )
    def run(table_hbm, idx_hbm, out_hbm, idx_v, rows_v, sem):
        wid = lax.axis_index("s") * mesh.num_cores + lax.axis_index("c")
        row0 = wid * rows_per_w

        @pl.loop(0, nch)
        def _chunk(c):
            r0 = row0 + c * G
            pltpu.sync_copy(idx_hbm.at[pl.ds(r0, G)], idx_v)
            copies = [
                pltpu.async_copy(
                    table_hbm.at[idx_v.at[j]],
                    rows_v.at[pl.ds(j * LANES, LANES)],
                    sem,
                )
                for j in range(G)
            ]
            for cp in copies:
                cp.wait()
            pltpu.sync_copy(rows_v, out_hbm.at[pl.ds(r0 * LANES, CHUNK)])

    out = run(table, idx2)
    return out.reshape(x.shape + (EMBED,))


# speedup vs baseline: 1.5763x; 1.0185x over previous
import functools

import jax
import jax.numpy as jnp
from jax import lax
from jax.experimental import pallas as pl
from jax.experimental.pallas import tpu as pltpu
from jax.experimental.pallas import tpu_sc as plsc

EMBED = 32
LANES = 128
G = 8
CHUNK = G * LANES


def kernel(x, table):
    idx = x.reshape(-1).astype(jnp.int32)
    n = idx.shape[0]
    idx2 = idx.reshape(n // LANES, LANES)

    mesh = plsc.VectorSubcoreMesh(core_axis_name="c", subcore_axis_name="s")
    nw = mesh.num_cores * mesh.num_subcores
    rows_per_w = (n // LANES) // nw
    nch = rows_per_w // G

    @functools.partial(
        pl.kernel,
        out_type=jax.ShapeDtypeStruct((n, EMBED), jnp.float32),
        mesh=mesh,
        scratch_types=[
            pltpu.VMEM((rows_per_w, LANES), jnp.int32),
            pltpu.VMEM((CHUNK, EMBED), jnp.float32),
            pltpu.VMEM((CHUNK, EMBED), jnp.float32),
            pltpu.SemaphoreType.DMA,
            pltpu.SemaphoreType.DMA,
        ],
        compiler_params=pltpu.CompilerParams(use_tc_tiling_on_sc=False),
    )
    def run(table_hbm, idx_hbm, out_hbm, idx_v, rows0, rows1, sem0, sem1):
        wid = lax.axis_index("s") * mesh.num_cores + lax.axis_index("c")
        row0 = wid * rows_per_w
        rows = (rows0, rows1)
        sems = (sem0, sem1)

        pltpu.sync_copy(idx_hbm.at[pl.ds(row0, rows_per_w)], idx_v)

        def fire(s, b):
            for j in range(G):
                pltpu.async_copy(
                    table_hbm.at[idx_v.at[s * G + j]],
                    rows[b].at[pl.ds(j * LANES, LANES)],
                    sems[b],
                )

        def drain_and_write(s, b):
            pltpu.make_async_copy(
                table_hbm.at[pl.ds(0, CHUNK)], rows[b], sems[b]
            ).wait()
            pltpu.sync_copy(
                rows[b], out_hbm.at[pl.ds((row0 + s * G) * LANES, CHUNK)]
            )

        @pl.loop(0, nch + 1, step=2)
        def _steps(c):
            for b in range(2):
                s = c + b

                @pl.when(s < nch)
                def _():
                    fire(s, b)

                @pl.when(jnp.logical_and(s > 0, s <= nch))
                def _():
                    drain_and_write(s - 1, 1 - b)

    out = run(table, idx2)
    return out.reshape(x.shape + (EMBED,))
